# baseline (speedup 1.0000x reference)
import jax
import jax.numpy as jnp
from jax.experimental import pallas as pl
from jax.experimental.pallas import tpu as pltpu

_K = 1024
_D = 64
_BETA = 0.25
_HW = 1024
_B = 16
_N = _B * _HW


_BB = 2


def _vq_block(z_ref, w_ref, wt_ref, out_ref, loss_ref):
    i = pl.program_id(0)
    w = w_ref[...]
    wt = wt_ref[...]
    w2 = jnp.sum(w ** 2, axis=1, keepdims=True)
    wt_hi = wt.astype(jnp.bfloat16)
    wt_lo = (wt - wt_hi.astype(jnp.float32)).astype(jnp.bfloat16)
    gdims = (((1,), (0,)), ((), ()))

    @pl.when(i == 0)
    def _init():
        loss_ref[...] = jnp.zeros_like(loss_ref)

    w2x = w + w
    for j in range(_BB):
        zd = z_ref[j]
        z2 = jnp.sum(zd ** 2, axis=0, keepdims=True)
        s2 = jax.lax.dot_general(
            w2x, zd, (((1,), (0,)), ((), ())),
            preferred_element_type=jnp.float32)
        d2 = (z2 + w2) - s2
        m = jnp.min(d2, axis=0, keepdims=True)
        iota = jax.lax.broadcasted_iota(jnp.int32, d2.shape, 0)
        cand = jnp.where(d2 == m, iota, _K)
        idx = jnp.min(cand, axis=0, keepdims=True)
        onehot = (cand == idx).astype(jnp.bfloat16)
        zq = (jax.lax.dot_general(wt_hi, onehot, gdims,
                                  preferred_element_type=jnp.float32)
              + jax.lax.dot_general(wt_lo, onehot, gdims,
                                    preferred_element_type=jnp.float32))
        out_ref[j] = zd + (zq - zd)
        loss_ref[...] += jnp.sum(m) * ((1.0 + _BETA) / (_N * _D))


def kernel(z, W):
    zr = z.reshape(_B, _D, _HW)
    Wt = W.T
    zq3, loss = pl.pallas_call(
        _vq_block,
        grid=(_B // _BB,),
        in_specs=[
            pl.BlockSpec((_BB, _D, _HW), lambda i: (i, 0, 0)),
            pl.BlockSpec((_K, _D), lambda i: (0, 0)),
            pl.BlockSpec((_D, _K), lambda i: (0, 0)),
        ],
        out_specs=[
            pl.BlockSpec((_BB, _D, _HW), lambda i: (i, 0, 0)),
            pl.BlockSpec((1, 1), lambda i: (0, 0)),
        ],
        out_shape=[
            jax.ShapeDtypeStruct((_B, _D, _HW), jnp.float32),
            jax.ShapeDtypeStruct((1, 1), jnp.float32),
        ],
    )(zr, W, Wt)
    return zq3.reshape(z.shape), loss[0, 0]
